# baseline (speedup 1.0000x reference)
import jax
import jax.numpy as jnp
from jax import lax
from jax.experimental import pallas as pl
from jax.experimental.pallas import tpu as pltpu
from jax.experimental.pallas import tpu_sc as plsc

M = 100000
B = 16384
D = 512
NC, NS = 2, 16
W = NC * NS
CM = 3136
CB = B // W
GC = 64
NG = CB // GC


def _wid():
    return lax.axis_index("s") * NC + lax.axis_index("c")


def _kernel_a_body(bx, by, bt, bidx, bstep, swap,
                   obx, oby, obt, obidx, obstep,
                   idxa, rows0, rows1, yv, tv, iv, sv,
                   cy, ct, ci, cs,
                   semc, sg0, sg1, sw0, sw1, semi, semic):
    w = _wid()
    lo = pl.multiple_of(jnp.minimum(w * CM, M - CM), 8)
    jlo = pl.multiple_of(w * CB, 8)

    bulk = [
        pltpu.async_copy(bx.at[pl.ds(lo, CM)], obx.at[pl.ds(lo, CM)], semc),
    ]
    iload = [
        pltpu.async_copy(by.at[pl.ds(lo, CM)], cy, semic),
        pltpu.async_copy(bt.at[pl.ds(lo, CM)], ct, semic),
        pltpu.async_copy(bidx.at[pl.ds(lo, CM)], ci, semic),
        pltpu.async_copy(bstep.at[pl.ds(lo, CM)], cs, semic),
    ]

    for c in range(NG):
        pltpu.sync_copy(swap.at[pl.ds(jlo + c * GC, GC)], idxa.at[c])

    for cp in iload:
        cp.wait()
    bulk.extend([
        pltpu.async_copy(cy, oby.at[pl.ds(lo, CM)], semc),
        pltpu.async_copy(ct, obt.at[pl.ds(lo, CM)], semc),
        pltpu.async_copy(ci, obidx.at[pl.ds(lo, CM)], semc),
        pltpu.async_copy(cs, obstep.at[pl.ds(lo, CM)], semc),
    ])

    ig = []
    for src, dstv in ((by, yv), (bt, tv), (bidx, iv), (bstep, sv)):
        for c in range(NG):
            ig.append(pltpu.async_copy(src.at[idxa.at[c]],
                                       dstv.at[pl.ds(c * GC, GC)], semi))

    rows = (rows0, rows1)
    sg = (sg0, sg1)
    sw = (sw0, sw1)
    g = [None, None]
    wb = [None, None]
    g[0] = pltpu.async_copy(bx.at[idxa.at[0]], rows[0], sg[0])
    for c in range(NG):
        b = c & 1
        nb = b ^ 1
        g[b].wait()
        if c + 1 < NG:
            if wb[nb] is not None:
                wb[nb].wait()
            g[nb] = pltpu.async_copy(bx.at[idxa.at[c + 1]], rows[nb], sg[nb])
        wb[b] = pltpu.async_copy(rows[b],
                                 obx.at[pl.ds(M + jlo + c * GC, GC)], sw[b])
    wb[0].wait()
    wb[1].wait()

    for cp in ig:
        cp.wait()
    pltpu.sync_copy(yv, oby.at[pl.ds(M + jlo, CB)])
    pltpu.sync_copy(tv, obt.at[pl.ds(M + jlo, CB)])
    pltpu.sync_copy(iv, obidx.at[pl.ds(M + jlo, CB)])
    pltpu.sync_copy(sv, obstep.at[pl.ds(M + jlo, CB)])

    for cp in bulk:
        cp.wait()


def _kernel_b_body(in_x, in_y, in_t, in_bidx, in_step, swap,
                   obx, oby, obt, obidx, obstep,
                   idxb, rows0, rows1, yv, tv, iv, sv,
                   sl0, sl1, ss0, ss1, semi, sems):
    w = _wid()
    jlo = pl.multiple_of(w * CB, 8)

    for c in range(NG):
        pltpu.sync_copy(swap.at[pl.ds(jlo + c * GC, GC)], idxb.at[c])

    lv = [
        pltpu.async_copy(in_y.at[pl.ds(jlo, CB)], yv, semi),
        pltpu.async_copy(in_t.at[pl.ds(jlo, CB)], tv, semi),
        pltpu.async_copy(in_bidx.at[pl.ds(jlo, CB)], iv, semi),
        pltpu.async_copy(in_step.at[pl.ds(jlo, CB)], sv, semi),
    ]

    rows = (rows0, rows1)
    sl = (sl0, sl1)
    ss = (ss0, ss1)
    ld = [None, None]
    sc = [None, None]
    ld[0] = pltpu.async_copy(in_x.at[pl.ds(jlo, GC)], rows[0], sl[0])
    for c in range(NG):
        b = c & 1
        nb = b ^ 1
        ld[b].wait()
        if c + 1 < NG:
            if sc[nb] is not None:
                sc[nb].wait()
            ld[nb] = pltpu.async_copy(in_x.at[pl.ds(jlo + (c + 1) * GC, GC)],
                                      rows[nb], sl[nb])
        sc[b] = pltpu.async_copy(rows[b], obx.at[idxb.at[c]], ss[b])
    sc[0].wait()
    sc[1].wait()

    for cp in lv:
        cp.wait()
    isc = []
    for srcv, dst in ((yv, oby), (tv, obt), (iv, obidx), (sv, obstep)):
        for c in range(NG):
            isc.append(pltpu.async_copy(srcv.at[pl.ds(c * GC, GC)],
                                        dst.at[idxb.at[c]], sems))
    for cp in isc:
        cp.wait()


def kernel(bx, by, bt, bidx, bstep, in_x, in_y, in_t, in_bidx, in_step, swap_idx):
    mesh = plsc.VectorSubcoreMesh(core_axis_name="c", subcore_axis_name="s")
    fsem = pltpu.SemaphoreType.DMA

    kernel_a = pl.kernel(
        _kernel_a_body,
        out_type=(
            jax.ShapeDtypeStruct((M + B, D), jnp.float32),
            jax.ShapeDtypeStruct((M + B,), jnp.int32),
            jax.ShapeDtypeStruct((M + B,), jnp.int32),
            jax.ShapeDtypeStruct((M + B,), jnp.int32),
            jax.ShapeDtypeStruct((M + B,), jnp.int32),
        ),
        mesh=mesh,
        scratch_types=[
            pltpu.VMEM((NG, GC), jnp.int32),
            pltpu.VMEM((GC, D), jnp.float32),
            pltpu.VMEM((GC, D), jnp.float32),
            pltpu.VMEM((CB,), jnp.int32),
            pltpu.VMEM((CB,), jnp.int32),
            pltpu.VMEM((CB,), jnp.int32),
            pltpu.VMEM((CB,), jnp.int32),
            pltpu.VMEM((CM,), jnp.int32),
            pltpu.VMEM((CM,), jnp.int32),
            pltpu.VMEM((CM,), jnp.int32),
            pltpu.VMEM((CM,), jnp.int32),
            fsem, fsem, fsem, fsem, fsem, fsem, fsem,
        ],
    )
    kernel_b = pl.kernel(
        _kernel_b_body,
        out_type=(),
        mesh=mesh,
        scratch_types=[
            pltpu.VMEM((NG, GC), jnp.int32),
            pltpu.VMEM((GC, D), jnp.float32),
            pltpu.VMEM((GC, D), jnp.float32),
            pltpu.VMEM((CB,), jnp.int32),
            pltpu.VMEM((CB,), jnp.int32),
            pltpu.VMEM((CB,), jnp.int32),
            pltpu.VMEM((CB,), jnp.int32),
            fsem, fsem, fsem, fsem, fsem, fsem,
        ],
    )

    outs = kernel_a(bx, by, bt, bidx, bstep, swap_idx)
    refs = [jax.new_ref(o) for o in outs]
    kernel_b(in_x, in_y, in_t, in_bidx, in_step, swap_idx, *refs)
    return tuple(r[...] for r in refs)


# speedup vs baseline: 23.1546x; 23.1546x over previous
import jax
import jax.numpy as jnp
from jax import lax
from jax.experimental import pallas as pl
from jax.experimental.pallas import tpu as pltpu
from jax.experimental.pallas import tpu_sc as plsc

M = 100000
B = 16384
D = 512
NC, NS = 2, 16
W = NC * NS
CB = B // W
GC = 64
NG = CB // GC

BLK = 2000
GRID = M // BLK


def _copy_body(xb, ox):
    ox[...] = xb[...]


def _int_copy_body(yb, tb, ib, sb, oy, ot, oi, osp):
    oy[pl.ds(0, M)] = yb[...]
    ot[pl.ds(0, M)] = tb[...]
    oi[pl.ds(0, M)] = ib[...]
    osp[pl.ds(0, M)] = sb[...]


def _sc_body(bx, by, bt, bidx, bstep, in_x, in_y, in_t, in_bidx, in_step, swap,
             obx, oby, obt, obidx, obstep,
             idxa, rows0, rows1, gy, gt, gi, gs, vy, vt, vi, vs,
             s0, s1, w0, w1, semg, semv, sems):
    w = lax.axis_index("s") * NC + lax.axis_index("c")
    jlo = pl.multiple_of(w * CB, 8)

    for c in range(NG):
        pltpu.sync_copy(swap.at[pl.ds(jlo + c * GC, GC)], idxa.at[c])

    lv = [
        pltpu.async_copy(in_y.at[pl.ds(jlo, CB)], vy, semv),
        pltpu.async_copy(in_t.at[pl.ds(jlo, CB)], vt, semv),
        pltpu.async_copy(in_bidx.at[pl.ds(jlo, CB)], vi, semv),
        pltpu.async_copy(in_step.at[pl.ds(jlo, CB)], vs, semv),
    ]
    ig = []
    for src, dstv in ((by, gy), (bt, gt), (bidx, gi), (bstep, gs)):
        for c in range(NG):
            ig.append(pltpu.async_copy(src.at[idxa.at[c]],
                                       dstv.at[pl.ds(c * GC, GC)], semg))

    rows = (rows0, rows1)
    sg = (s0, s1)
    sw = (w0, w1)
    g = [None, None]
    wb = [None, None]
    g[0] = pltpu.async_copy(bx.at[idxa.at[0]], rows[0], sg[0])
    for c in range(NG):
        b = c & 1
        nb = b ^ 1
        g[b].wait()
        if c + 1 < NG:
            if wb[nb] is not None:
                wb[nb].wait()
            g[nb] = pltpu.async_copy(bx.at[idxa.at[c + 1]], rows[nb], sg[nb])
        wb[b] = pltpu.async_copy(rows[b],
                                 obx.at[pl.ds(M + jlo + c * GC, GC)], sw[b])
    wb[0].wait()
    wb[1].wait()

    ld = [None, None]
    sc = [None, None]
    ld[0] = pltpu.async_copy(in_x.at[pl.ds(jlo, GC)], rows[0], sg[0])
    for c in range(NG):
        b = c & 1
        nb = b ^ 1
        ld[b].wait()
        if c + 1 < NG:
            if sc[nb] is not None:
                sc[nb].wait()
            ld[nb] = pltpu.async_copy(in_x.at[pl.ds(jlo + (c + 1) * GC, GC)],
                                      rows[nb], sg[nb])
        sc[b] = pltpu.async_copy(rows[b], obx.at[idxa.at[c]], sw[b])
    sc[0].wait()
    sc[1].wait()

    for cp in ig:
        cp.wait()
    pltpu.sync_copy(gy, oby.at[pl.ds(M + jlo, CB)])
    pltpu.sync_copy(gt, obt.at[pl.ds(M + jlo, CB)])
    pltpu.sync_copy(gi, obidx.at[pl.ds(M + jlo, CB)])
    pltpu.sync_copy(gs, obstep.at[pl.ds(M + jlo, CB)])

    for cp in lv:
        cp.wait()
    isc = []
    for srcv, dst in ((vy, oby), (vt, obt), (vi, obidx), (vs, obstep)):
        for c in range(NG):
            isc.append(pltpu.async_copy(srcv.at[pl.ds(c * GC, GC)],
                                        dst.at[idxa.at[c]], sems))
    for cp in isc:
        cp.wait()


def kernel(bx, by, bt, bidx, bstep, in_x, in_y, in_t, in_bidx, in_step, swap_idx):
    copy_kernel = pl.pallas_call(
        _copy_body,
        grid=(GRID,),
        in_specs=[pl.BlockSpec((BLK, D), lambda i: (i, 0))],
        out_specs=pl.BlockSpec((BLK, D), lambda i: (i, 0)),
        out_shape=jax.ShapeDtypeStruct((M + B, D), jnp.float32),
    )
    int_copy_kernel = pl.pallas_call(
        _int_copy_body,
        out_shape=(
            jax.ShapeDtypeStruct((M + B,), jnp.int32),
            jax.ShapeDtypeStruct((M + B,), jnp.int32),
            jax.ShapeDtypeStruct((M + B,), jnp.int32),
            jax.ShapeDtypeStruct((M + B,), jnp.int32),
        ),
    )

    mesh = plsc.VectorSubcoreMesh(core_axis_name="c", subcore_axis_name="s")
    fsem = pltpu.SemaphoreType.DMA
    sc_kernel = pl.kernel(
        _sc_body,
        out_type=(),
        mesh=mesh,
        scratch_types=[
            pltpu.VMEM((NG, GC), jnp.int32),
            pltpu.VMEM((GC, D), jnp.float32),
            pltpu.VMEM((GC, D), jnp.float32),
            pltpu.VMEM((CB,), jnp.int32),
            pltpu.VMEM((CB,), jnp.int32),
            pltpu.VMEM((CB,), jnp.int32),
            pltpu.VMEM((CB,), jnp.int32),
            pltpu.VMEM((CB,), jnp.int32),
            pltpu.VMEM((CB,), jnp.int32),
            pltpu.VMEM((CB,), jnp.int32),
            pltpu.VMEM((CB,), jnp.int32),
            fsem, fsem, fsem, fsem, fsem, fsem, fsem,
        ],
    )

    ox = copy_kernel(bx)
    oints = int_copy_kernel(by, bt, bidx, bstep)
    refs = [jax.new_ref(o) for o in (ox, *oints)]
    sc_kernel(bx, by, bt, bidx, bstep,
              in_x, in_y, in_t, in_bidx, in_step, swap_idx, *refs)
    return tuple(r[...] for r in refs)


# speedup vs baseline: 23.3468x; 1.0083x over previous
import jax
import jax.numpy as jnp
from jax import lax
from jax.experimental import pallas as pl
from jax.experimental.pallas import tpu as pltpu
from jax.experimental.pallas import tpu_sc as plsc

M = 100000
B = 16384
D = 512
NC, NS = 2, 16
W = NC * NS
CB = B // W
GC = 32
NG = CB // GC

BLK = 4000
GRID = M // BLK


def _copy_body(xb, ox):
    ox[...] = xb[...]


def _int_copy_body(yb, tb, ib, sb, oy, ot, oi, osp):
    oy[pl.ds(0, M)] = yb[...]
    ot[pl.ds(0, M)] = tb[...]
    oi[pl.ds(0, M)] = ib[...]
    osp[pl.ds(0, M)] = sb[...]


def _sc_body(bx, by, bt, bidx, bstep, in_x, in_y, in_t, in_bidx, in_step, swap,
             obx, oby, obt, obidx, obstep,
             idxa, g0, g1, r0, r1, gy, gt, gi, gs, vy, vt, vi, vs,
             sga, sgb, ssa, ssb, semg, semv, sems):
    w = lax.axis_index("s") * NC + lax.axis_index("c")
    jlo = pl.multiple_of(w * CB, 8)

    for c in range(NG):
        pltpu.sync_copy(swap.at[pl.ds(jlo + c * GC, GC)], idxa.at[c])

    lv = [
        pltpu.async_copy(in_y.at[pl.ds(jlo, CB)], vy, semv),
        pltpu.async_copy(in_t.at[pl.ds(jlo, CB)], vt, semv),
        pltpu.async_copy(in_bidx.at[pl.ds(jlo, CB)], vi, semv),
        pltpu.async_copy(in_step.at[pl.ds(jlo, CB)], vs, semv),
    ]
    ig = []
    for src, dstv in ((by, gy), (bt, gt), (bidx, gi), (bstep, gs)):
        for c in range(NG):
            ig.append(pltpu.async_copy(src.at[idxa.at[c]],
                                       dstv.at[pl.ds(c * GC, GC)], semg))

    grows = (g0, g1)
    srows = (r0, r1)
    gl = [None, None]
    gst = [None, None]
    sl = [None, None]
    sst = [None, None]
    gl[0] = pltpu.async_copy(bx.at[idxa.at[0]], grows[0], sga)
    sl[0] = pltpu.async_copy(in_x.at[pl.ds(jlo, GC)], srows[0], ssa)
    for c in range(NG):
        b = c & 1
        nb = b ^ 1
        gl[b].wait()
        sl[b].wait()
        if c + 1 < NG:
            if gst[nb] is not None:
                gst[nb].wait()
                sst[nb].wait()
            gl[nb] = pltpu.async_copy(bx.at[idxa.at[c + 1]], grows[nb], sga)
            sl[nb] = pltpu.async_copy(in_x.at[pl.ds(jlo + (c + 1) * GC, GC)],
                                      srows[nb], ssa)
        gst[b] = pltpu.async_copy(grows[b],
                                  obx.at[pl.ds(M + jlo + c * GC, GC)], sgb)
        sst[b] = pltpu.async_copy(srows[b], obx.at[idxa.at[c]], ssb)
    gst[0].wait()
    sst[0].wait()
    gst[1].wait()
    sst[1].wait()

    for cp in ig:
        cp.wait()
    pltpu.sync_copy(gy, oby.at[pl.ds(M + jlo, CB)])
    pltpu.sync_copy(gt, obt.at[pl.ds(M + jlo, CB)])
    pltpu.sync_copy(gi, obidx.at[pl.ds(M + jlo, CB)])
    pltpu.sync_copy(gs, obstep.at[pl.ds(M + jlo, CB)])

    for cp in lv:
        cp.wait()
    isc = []
    for srcv, dst in ((vy, oby), (vt, obt), (vi, obidx), (vs, obstep)):
        for c in range(NG):
            isc.append(pltpu.async_copy(srcv.at[pl.ds(c * GC, GC)],
                                        dst.at[idxa.at[c]], sems))
    for cp in isc:
        cp.wait()


def kernel(bx, by, bt, bidx, bstep, in_x, in_y, in_t, in_bidx, in_step, swap_idx):
    copy_kernel = pl.pallas_call(
        _copy_body,
        grid=(GRID,),
        in_specs=[pl.BlockSpec((BLK, D), lambda i: (i, 0))],
        out_specs=pl.BlockSpec((BLK, D), lambda i: (i, 0)),
        out_shape=jax.ShapeDtypeStruct((M + B, D), jnp.float32),
    )
    int_copy_kernel = pl.pallas_call(
        _int_copy_body,
        out_shape=(
            jax.ShapeDtypeStruct((M + B,), jnp.int32),
            jax.ShapeDtypeStruct((M + B,), jnp.int32),
            jax.ShapeDtypeStruct((M + B,), jnp.int32),
            jax.ShapeDtypeStruct((M + B,), jnp.int32),
        ),
    )

    mesh = plsc.VectorSubcoreMesh(core_axis_name="c", subcore_axis_name="s")
    fsem = pltpu.SemaphoreType.DMA
    sc_kernel = pl.kernel(
        _sc_body,
        out_type=(),
        mesh=mesh,
        scratch_types=[
            pltpu.VMEM((NG, GC), jnp.int32),
            pltpu.VMEM((GC, D), jnp.float32),
            pltpu.VMEM((GC, D), jnp.float32),
            pltpu.VMEM((GC, D), jnp.float32),
            pltpu.VMEM((GC, D), jnp.float32),
            pltpu.VMEM((CB,), jnp.int32),
            pltpu.VMEM((CB,), jnp.int32),
            pltpu.VMEM((CB,), jnp.int32),
            pltpu.VMEM((CB,), jnp.int32),
            pltpu.VMEM((CB,), jnp.int32),
            pltpu.VMEM((CB,), jnp.int32),
            pltpu.VMEM((CB,), jnp.int32),
            pltpu.VMEM((CB,), jnp.int32),
            fsem, fsem, fsem, fsem, fsem, fsem, fsem,
        ],
    )

    ox = copy_kernel(bx)
    oints = int_copy_kernel(by, bt, bidx, bstep)
    refs = [jax.new_ref(o) for o in (ox, *oints)]
    sc_kernel(bx, by, bt, bidx, bstep,
              in_x, in_y, in_t, in_bidx, in_step, swap_idx, *refs)
    return tuple(r[...] for r in refs)


# speedup vs baseline: 23.3928x; 1.0020x over previous
import jax
import jax.numpy as jnp
from jax import lax
from jax.experimental import pallas as pl
from jax.experimental.pallas import tpu as pltpu
from jax.experimental.pallas import tpu_sc as plsc

M = 100000
B = 16384
D = 512
NC, NS = 2, 16
W = NC * NS
CB = B // W
GC = 32
NG = CB // GC
NBUF = 3

BLK = 4000
GRID = M // BLK


def _copy_body(xb, ox):
    ox[...] = xb[...]


def _int_copy_body(yb, tb, ib, sb, oy, ot, oi, osp):
    oy[pl.ds(0, M)] = yb[...]
    ot[pl.ds(0, M)] = tb[...]
    oi[pl.ds(0, M)] = ib[...]
    osp[pl.ds(0, M)] = sb[...]


def _sc_body(bx, by, bt, bidx, bstep, in_x, in_y, in_t, in_bidx, in_step, swap,
             obx, oby, obt, obidx, obstep,
             idxa, idxf, g0, g1, g2, r0, r1, r2,
             gy, gt, gi, gs, vy, vt, vi, vs,
             sga, sgb, ssa, ssb, semg, semv, sems):
    w = lax.axis_index("s") * NC + lax.axis_index("c")
    jlo = pl.multiple_of(w * CB, 8)

    for c in range(NG):
        pltpu.sync_copy(swap.at[pl.ds(jlo + c * GC, GC)], idxa.at[c])
    pltpu.sync_copy(swap.at[pl.ds(jlo, CB)], idxf)

    lv = [
        pltpu.async_copy(in_y.at[pl.ds(jlo, CB)], vy, semv),
        pltpu.async_copy(in_t.at[pl.ds(jlo, CB)], vt, semv),
        pltpu.async_copy(in_bidx.at[pl.ds(jlo, CB)], vi, semv),
        pltpu.async_copy(in_step.at[pl.ds(jlo, CB)], vs, semv),
    ]
    ig = [
        pltpu.async_copy(by.at[idxf], gy, semg),
        pltpu.async_copy(bt.at[idxf], gt, semg),
        pltpu.async_copy(bidx.at[idxf], gi, semg),
        pltpu.async_copy(bstep.at[idxf], gs, semg),
    ]

    grows = (g0, g1, g2)
    srows = (r0, r1, r2)
    gl = [None, None, None]
    gst = [None, None, None]
    sl = [None, None, None]
    sst = [None, None, None]
    for c in range(2):
        gl[c] = pltpu.async_copy(bx.at[idxa.at[c]], grows[c], sga)
        sl[c] = pltpu.async_copy(in_x.at[pl.ds(jlo + c * GC, GC)],
                                 srows[c], ssa)
    for c in range(NG):
        b = c % NBUF
        nx = (c + 2) % NBUF
        gl[b].wait()
        sl[b].wait()
        if c + 2 < NG:
            if gst[nx] is not None:
                gst[nx].wait()
                sst[nx].wait()
            gl[nx] = pltpu.async_copy(bx.at[idxa.at[c + 2]], grows[nx], sga)
            sl[nx] = pltpu.async_copy(in_x.at[pl.ds(jlo + (c + 2) * GC, GC)],
                                      srows[nx], ssa)
        gst[b] = pltpu.async_copy(grows[b],
                                  obx.at[pl.ds(M + jlo + c * GC, GC)], sgb)
        sst[b] = pltpu.async_copy(srows[b], obx.at[idxa.at[c]], ssb)
    for b in range(NBUF):
        gst[b].wait()
        sst[b].wait()

    for cp in ig:
        cp.wait()
    pltpu.sync_copy(gy, oby.at[pl.ds(M + jlo, CB)])
    pltpu.sync_copy(gt, obt.at[pl.ds(M + jlo, CB)])
    pltpu.sync_copy(gi, obidx.at[pl.ds(M + jlo, CB)])
    pltpu.sync_copy(gs, obstep.at[pl.ds(M + jlo, CB)])

    for cp in lv:
        cp.wait()
    isc = [
        pltpu.async_copy(vy, oby.at[idxf], sems),
        pltpu.async_copy(vt, obt.at[idxf], sems),
        pltpu.async_copy(vi, obidx.at[idxf], sems),
        pltpu.async_copy(vs, obstep.at[idxf], sems),
    ]
    for cp in isc:
        cp.wait()


def kernel(bx, by, bt, bidx, bstep, in_x, in_y, in_t, in_bidx, in_step, swap_idx):
    copy_kernel = pl.pallas_call(
        _copy_body,
        grid=(GRID,),
        in_specs=[pl.BlockSpec((BLK, D), lambda i: (i, 0))],
        out_specs=pl.BlockSpec((BLK, D), lambda i: (i, 0)),
        out_shape=jax.ShapeDtypeStruct((M + B, D), jnp.float32),
    )
    int_copy_kernel = pl.pallas_call(
        _int_copy_body,
        out_shape=(
            jax.ShapeDtypeStruct((M + B,), jnp.int32),
            jax.ShapeDtypeStruct((M + B,), jnp.int32),
            jax.ShapeDtypeStruct((M + B,), jnp.int32),
            jax.ShapeDtypeStruct((M + B,), jnp.int32),
        ),
    )

    mesh = plsc.VectorSubcoreMesh(core_axis_name="c", subcore_axis_name="s")
    fsem = pltpu.SemaphoreType.DMA
    sc_kernel = pl.kernel(
        _sc_body,
        out_type=(),
        mesh=mesh,
        scratch_types=[
            pltpu.VMEM((NG, GC), jnp.int32),
            pltpu.VMEM((CB,), jnp.int32),
            pltpu.VMEM((GC, D), jnp.float32),
            pltpu.VMEM((GC, D), jnp.float32),
            pltpu.VMEM((GC, D), jnp.float32),
            pltpu.VMEM((GC, D), jnp.float32),
            pltpu.VMEM((GC, D), jnp.float32),
            pltpu.VMEM((GC, D), jnp.float32),
            pltpu.VMEM((CB,), jnp.int32),
            pltpu.VMEM((CB,), jnp.int32),
            pltpu.VMEM((CB,), jnp.int32),
            pltpu.VMEM((CB,), jnp.int32),
            pltpu.VMEM((CB,), jnp.int32),
            pltpu.VMEM((CB,), jnp.int32),
            pltpu.VMEM((CB,), jnp.int32),
            pltpu.VMEM((CB,), jnp.int32),
            fsem, fsem, fsem, fsem, fsem, fsem, fsem,
        ],
    )

    ox = copy_kernel(bx)
    oints = int_copy_kernel(by, bt, bidx, bstep)
    refs = [jax.new_ref(o) for o in (ox, *oints)]
    sc_kernel(bx, by, bt, bidx, bstep,
              in_x, in_y, in_t, in_bidx, in_step, swap_idx, *refs)
    return tuple(r[...] for r in refs)
